# baseline (speedup 1.0000x reference)
import functools

import jax
import jax.numpy as jnp
from jax import lax
from jax.experimental import pallas as pl
from jax.experimental.pallas import tpu as pltpu
from jax.experimental.pallas import tpu_sc as plsc

VOCAB = 1000000
EMBED_DIM = 64
BATCH = 16384
_HALF = EMBED_DIM // 2

_NUM_CORES = 2
_NUM_SUBCORES = 16
_NUM_WORKERS = _NUM_CORES * _NUM_SUBCORES
_B_PER_W = BATCH // _NUM_WORKERS
_ICHUNK = 128
_N_ICHUNKS = _B_PER_W // _ICHUNK


def _make_sc_gather():
    mesh = plsc.VectorSubcoreMesh(core_axis_name="c", subcore_axis_name="s")

    @functools.partial(
        pl.kernel,
        mesh=mesh,
        out_type=jax.ShapeDtypeStruct((BATCH, EMBED_DIM), jnp.float32),
        scratch_types=[
            pltpu.VMEM((_N_ICHUNKS, _ICHUNK), jnp.int32),
            pltpu.VMEM((_B_PER_W, _HALF), jnp.float32),
            pltpu.VMEM((_B_PER_W, _HALF), jnp.float32),
            pltpu.SemaphoreType.DMA,
        ],
        compiler_params=pltpu.CompilerParams(use_tc_tiling_on_sc=False),
    )
    def k(idx_hbm, t0_hbm, t1_hbm, out_hbm, idx_v, r0_v, r1_v, sem):
        wid = lax.axis_index("s") * _NUM_CORES + lax.axis_index("c")
        wbase = wid * _B_PER_W
        for c in range(_N_ICHUNKS):
            pltpu.sync_copy(
                idx_hbm.at[pl.ds(wbase + c * _ICHUNK, _ICHUNK)], idx_v.at[c]
            )
        for c in range(_N_ICHUNKS):
            pltpu.async_copy(
                t0_hbm.at[idx_v.at[c]],
                r0_v.at[pl.ds(c * _ICHUNK, _ICHUNK)],
                sem,
            )
            pltpu.async_copy(
                t1_hbm.at[idx_v.at[c]],
                r1_v.at[pl.ds(c * _ICHUNK, _ICHUNK)],
                sem,
            )
        for c in range(2 * _N_ICHUNKS):
            pltpu.make_async_copy(
                t0_hbm.at[idx_v.at[0]],
                r0_v.at[pl.ds(0, _ICHUNK)],
                sem,
            ).wait()
        pltpu.sync_copy(r0_v, out_hbm.at[pl.ds(wbase, _B_PER_W), pl.ds(0, _HALF)])
        pltpu.sync_copy(
            r1_v, out_hbm.at[pl.ds(wbase, _B_PER_W), pl.ds(_HALF, _HALF)]
        )

    return k


_sc_gather = _make_sc_gather()


@jax.jit
def kernel(category, table):
    t0 = table[:, :_HALF]
    t1 = table[:, _HALF:]
    return _sc_gather(category.astype(jnp.int32), t0, t1)


# speedup vs baseline: 5.7772x; 5.7772x over previous
import functools

import jax
import jax.numpy as jnp
from jax import lax
from jax.experimental import pallas as pl
from jax.experimental.pallas import tpu as pltpu
from jax.experimental.pallas import tpu_sc as plsc

VOCAB = 1000000
EMBED_DIM = 64
BATCH = 16384

_NUM_CORES = 2
_NUM_SUBCORES = 16
_NUM_WORKERS = _NUM_CORES * _NUM_SUBCORES
_LANES = 16
_QB = 8
_NTILES = 7813
_SLAB_WORDS = _NTILES * 1024
_TPW = 244
_CT = 4
_CHUNK_WORDS = _QB * _CT * 1024
_MAXM = BATCH


def _make_sc_gather():
    mesh = plsc.VectorSubcoreMesh(core_axis_name="c", subcore_axis_name="s")

    @functools.partial(
        pl.kernel,
        mesh=mesh,
        out_type=jax.ShapeDtypeStruct((BATCH * EMBED_DIM,), jnp.float32),
        scratch_types=[
            pltpu.VMEM((BATCH + _LANES,), jnp.int32),
            pltpu.VMEM((_MAXM + _LANES,), jnp.int32),
            pltpu.VMEM((_MAXM + _LANES,), jnp.int32),
            pltpu.VMEM((_MAXM + _LANES,), jnp.int32),
            pltpu.VMEM((_MAXM + _LANES,), jnp.int32),
            pltpu.VMEM((_CHUNK_WORDS,), jnp.float32),
            pltpu.VMEM((_LANES * EMBED_DIM,), jnp.float32),
            pltpu.SemaphoreType.DMA,
            pltpu.SemaphoreType.DMA,
        ],
        compiler_params=pltpu.CompilerParams(needs_layout_passes=False),
    )
    def k(idx_hbm, tab_t_hbm, out_hbm, idx_v, mv_v, mp_v, cv_v, cp_v,
          chunk_v, stage_v, sem_in, sem_out):
        tbl3 = tab_t_hbm.reshape(_QB, _QB, VOCAB)
        wid = lax.axis_index("s") * _NUM_CORES + lax.axis_index("c")

        for c in range(BATCH // 2048):
            pltpu.sync_copy(
                idx_hbm.at[pl.ds(c * 2048, 2048)], idx_v.at[pl.ds(c * 2048, 2048)]
            )

        lo = wid * (_TPW * 128)
        hi = jnp.where(wid == _NUM_WORKERS - 1, VOCAB, lo + _TPW * 128)
        lov = jnp.full((_LANES,), lo, jnp.int32)
        hiv = jnp.full((_LANES,), hi, jnp.int32)
        iota = lax.iota(jnp.int32, _LANES)

        def p1(g, o):
            iv = idx_v[pl.ds(g * _LANES, _LANES)]
            posv = iota + g * _LANES
            m = jnp.logical_and(iv >= lov, iv < hiv)
            plsc.store_compressed(mv_v.at[pl.ds(o, _LANES)], iv, mask=m)
            plsc.store_compressed(mp_v.at[pl.ds(o, _LANES)], posv, mask=m)
            cnt = plsc.all_reduce_population_count(m)
            return o + cnt[0]

        nmine = lax.fori_loop(0, BATCH // _LANES, p1, jnp.int32(0))

        tbase = wid * _TPW
        nch = jnp.where(wid == _NUM_WORKERS - 1,
                        (_NTILES - 31 * _TPW + _CT - 1) // _CT,
                        (_TPW + _CT - 1) // _CT)
        nout = jnp.int32(0)

        def chunk_body(c, nout):
            t0 = jnp.minimum(tbase + c * _CT, _NTILES - _CT)
            vlo = t0 * 128
            vlov = jnp.broadcast_to(vlo, (_LANES,)).astype(jnp.int32)
            vhiv = vlov + _CT * 128
            for q in range(_QB):
                pltpu.async_copy(
                    tbl3.at[q, 0].at[pl.ds(t0 * 1024, _CT * 1024)],
                    chunk_v.at[pl.ds(q * _CT * 1024, _CT * 1024)],
                    sem_in,
                )
            for q in range(_QB):
                pltpu.make_async_copy(
                    tbl3.at[0, 0].at[pl.ds(0, _CT * 1024)],
                    chunk_v.at[pl.ds(0, _CT * 1024)],
                    sem_in,
                ).wait()

            def rc(g, o2):
                mvs = mv_v[pl.ds(g * _LANES, _LANES)]
                mps = mp_v[pl.ds(g * _LANES, _LANES)]
                valid = (iota + g * _LANES) < nmine
                m = jnp.logical_and(
                    jnp.logical_and(mvs >= vlov, mvs < vhiv), valid
                )
                plsc.store_compressed(cv_v.at[pl.ds(o2, _LANES)], mvs, mask=m)
                plsc.store_compressed(cp_v.at[pl.ds(o2, _LANES)], mps, mask=m)
                cnt = plsc.all_reduce_population_count(m)
                return o2 + cnt[0]

            ngrp_mine = (nmine + _LANES - 1) // _LANES
            nthis = lax.fori_loop(0, ngrp_mine, rc, jnp.int32(0))

            def grp(g2, nout):
                cvs = cv_v[pl.ds(g2 * _LANES, _LANES)]
                cps = cp_v[pl.ds(g2 * _LANES, _LANES)]
                valid = (iota + g2 * _LANES) < nthis
                offv = jnp.where(
                    valid,
                    (lax.shift_right_logical(cvs - vlo, 7)) * 1024
                    + lax.bitwise_and(cvs, 127),
                    0,
                )
                for f in range(EMBED_DIM):
                    fc = (f // 8) * (_CT * 1024) + (f % 8) * 128
                    vals = plsc.load_gather(chunk_v, [offv + fc])
                    plsc.store_scatter(
                        stage_v, [iota * EMBED_DIM + f], vals
                    )
                for l in range(_LANES):
                    @pl.when(g2 * _LANES + l < nthis)
                    def _():
                        pos = cps[l]
                        pltpu.async_copy(
                            stage_v.at[pl.ds(l * EMBED_DIM, EMBED_DIM)],
                            out_hbm.at[pl.ds(pos * EMBED_DIM, EMBED_DIM)],
                            sem_out,
                        )
                def dr(j, carry):
                    pltpu.make_async_copy(
                        stage_v.at[pl.ds(0, EMBED_DIM)],
                        out_hbm.at[pl.ds(0, EMBED_DIM)],
                        sem_out,
                    ).wait()
                    return carry

                nhere = jnp.minimum(nthis - g2 * _LANES, _LANES)
                lax.fori_loop(0, nhere, dr, 0)
                return nout + nhere

            ngrp_this = (nthis + _LANES - 1) // _LANES
            return lax.fori_loop(0, ngrp_this, grp, nout)

        lax.fori_loop(0, nch, chunk_body, nout)

    return k


_sc_gather = _make_sc_gather()


@jax.jit
def kernel(category, table):
    flat = _sc_gather(category.astype(jnp.int32), table.T)
    return flat.reshape(BATCH, EMBED_DIM)


# speedup vs baseline: 8.2178x; 1.4225x over previous
import functools

import jax
import jax.numpy as jnp
from jax import lax
from jax.experimental import pallas as pl
from jax.experimental.pallas import tpu as pltpu
from jax.experimental.pallas import tpu_sc as plsc

VOCAB = 1000000
EMBED_DIM = 64
BATCH = 16384

_NUM_CORES = 2
_NUM_SUBCORES = 16
_NUM_WORKERS = _NUM_CORES * _NUM_SUBCORES
_LANES = 16
_QB = 8
_NTILES = 7813
_TPW = 244
_CT = 4
_CW = _CT * 1024
_CHUNK_WORDS = _QB * _CW
_NCH = 64
_MAXM = BATCH
_POS_SHIFT = 15


def _make_sc_gather():
    mesh = plsc.VectorSubcoreMesh(core_axis_name="c", subcore_axis_name="s")

    @functools.partial(
        pl.kernel,
        mesh=mesh,
        out_type=jax.ShapeDtypeStruct((BATCH * EMBED_DIM,), jnp.float32),
        scratch_types=[
            pltpu.VMEM((BATCH + _LANES,), jnp.int32),
            pltpu.VMEM((_MAXM + _LANES,), jnp.int32),
            pltpu.VMEM((_MAXM + _LANES,), jnp.int32),
            pltpu.VMEM((_CHUNK_WORDS,), jnp.float32),
            pltpu.VMEM((_CHUNK_WORDS,), jnp.float32),
            pltpu.VMEM((_LANES * EMBED_DIM,), jnp.float32),
            pltpu.SemaphoreType.DMA,
            pltpu.SemaphoreType.DMA,
            pltpu.SemaphoreType.DMA,
        ],
        compiler_params=pltpu.CompilerParams(needs_layout_passes=False),
    )
    def k(idx_hbm, tab_t_hbm, out_hbm, idx_v, m_v, c_v,
          bufa_v, bufb_v, stage_v, sem_a, sem_b, sem_out):
        tbl3 = tab_t_hbm.reshape(_QB, _QB, VOCAB)
        wid = lax.axis_index("s") * _NUM_CORES + lax.axis_index("c")
        iota = lax.iota(jnp.int32, _LANES)

        for c in range(BATCH // 2048):
            pltpu.sync_copy(
                idx_hbm.at[pl.ds(c * 2048, 2048)],
                idx_v.at[pl.ds(c * 2048, 2048)],
            )

        lo = wid * (_TPW * 128)
        hi = jnp.where(wid == _NUM_WORKERS - 1, VOCAB, lo + _TPW * 128)
        lov = jnp.full((_LANES,), lo, jnp.int32)
        hiv = jnp.full((_LANES,), hi, jnp.int32)

        def p1(g, o):
            iv = idx_v[pl.ds(g * _LANES, _LANES)]
            posv = iota + g * _LANES
            m = jnp.logical_and(iv >= lov, iv < hiv)
            pk = (iv - lov) + lax.shift_left(posv, _POS_SHIFT)
            plsc.store_compressed(m_v.at[pl.ds(o, _LANES)], pk, mask=m)
            cnt = plsc.all_reduce_population_count(m)
            return o + cnt[0]

        nmine = lax.fori_loop(0, BATCH // _LANES, p1, jnp.int32(0))
        ngrp_mine = (nmine + _LANES - 1) // _LANES
        tbase = wid * _TPW
        relmask = jnp.full((_LANES,), (1 << _POS_SHIFT) - 1, jnp.int32)

        def fire_chunk(c, buf, sem):
            t0 = jnp.minimum(tbase + c * _CT, _NTILES - _CT)
            for q in range(_QB):
                pltpu.async_copy(
                    tbl3.at[q, 0].at[pl.ds(t0 * 1024, _CW)],
                    buf.at[pl.ds(q * _CW, _CW)],
                    sem,
                )

        def wait_chunk(buf, sem):
            for q in range(_QB):
                pltpu.make_async_copy(
                    tbl3.at[0, 0].at[pl.ds(0, _CW)],
                    buf.at[pl.ds(0, _CW)],
                    sem,
                ).wait()

        def process(c, buf, nout):
            t0 = jnp.minimum(tbase + c * _CT, _NTILES - _CT)
            rlo = t0 * 128 - lo
            rlov = jnp.broadcast_to(rlo, (_LANES,)).astype(jnp.int32)
            rhiv = rlov + _CT * 128

            def rc(g, o2):
                pks = m_v[pl.ds(g * _LANES, _LANES)]
                rel = lax.bitwise_and(pks, relmask)
                valid = (iota + g * _LANES) < nmine
                m = jnp.logical_and(
                    jnp.logical_and(rel >= rlov, rel < rhiv), valid
                )
                plsc.store_compressed(c_v.at[pl.ds(o2, _LANES)], pks, mask=m)
                cnt = plsc.all_reduce_population_count(m)
                return o2 + cnt[0]

            nthis = lax.fori_loop(0, ngrp_mine, rc, jnp.int32(0))

            def grp(g2, nout):
                pks = c_v[pl.ds(g2 * _LANES, _LANES)]
                rel = lax.bitwise_and(pks, relmask)
                vic = rel - rlov
                valid = (iota + g2 * _LANES) < nthis
                offv = jnp.where(
                    valid,
                    lax.shift_left(lax.shift_right_logical(vic, 7), 10)
                    + lax.bitwise_and(vic, 127),
                    0,
                )
                for f in range(EMBED_DIM):
                    fc = (f // 8) * _CW + (f % 8) * 128
                    vals = plsc.load_gather(chunk_ref := buf, [offv + fc])
                    plsc.store_scatter(stage_v, [iota * EMBED_DIM + f], vals)
                for l in range(_LANES):
                    @pl.when(g2 * _LANES + l < nthis)
                    def _():
                        pos = lax.shift_right_logical(pks[l], _POS_SHIFT)
                        pltpu.async_copy(
                            stage_v.at[pl.ds(l * EMBED_DIM, EMBED_DIM)],
                            out_hbm.at[pl.ds(pos * EMBED_DIM, EMBED_DIM)],
                            sem_out,
                        )

                def dr(j, carry):
                    pltpu.make_async_copy(
                        stage_v.at[pl.ds(0, EMBED_DIM)],
                        out_hbm.at[pl.ds(0, EMBED_DIM)],
                        sem_out,
                    ).wait()
                    return carry

                nhere = jnp.minimum(nthis - g2 * _LANES, _LANES)
                lax.fori_loop(0, nhere, dr, 0)
                return nout + nhere

            ngrp_this = (nthis + _LANES - 1) // _LANES
            return lax.fori_loop(0, ngrp_this, grp, nout)

        fire_chunk(jnp.int32(0), bufa_v, sem_a)

        def pair(cc, nout):
            c0 = 2 * cc
            fire_chunk(c0 + 1, bufb_v, sem_b)
            wait_chunk(bufa_v, sem_a)
            nout = process(c0, bufa_v, nout)
            fire_chunk(c0 + 2, bufa_v, sem_a)
            wait_chunk(bufb_v, sem_b)
            nout = process(c0 + 1, bufb_v, nout)
            return nout

        lax.fori_loop(0, _NCH // 2, pair, jnp.int32(0))
        wait_chunk(bufa_v, sem_a)

    return k


_sc_gather = _make_sc_gather()


@jax.jit
def kernel(category, table):
    flat = _sc_gather(category.astype(jnp.int32), table.T)
    return flat.reshape(BATCH, EMBED_DIM)
